# Initial kernel scaffold; baseline (speedup 1.0000x reference)
import jax
import jax.numpy as jnp
from jax.experimental import pallas as pl


def kernel(x, edge_index, W1, b1, W2, b2):
    raise NotImplementedError("write your pallas kernel here")


# speedup vs baseline: 33.4844x; 33.4844x over previous
import functools

import jax
import jax.numpy as jnp
from jax import lax
from jax.experimental import pallas as pl
from jax.experimental.pallas import tpu as pltpu
from jax.experimental.pallas import tpu_sc as plsc

N_NODES = 10000
N_EDGES = 320000
IN_DIM = 128
HID = 16
OUT_DIM = 7

NC = 2
NS = 16
NW = NC * NS
LANES = 16
CHUNK = 128
K = 79
E_PAD = NW * K * CHUNK
N_PAD = 10240
RPS = N_PAD // NS
TRASH = N_PAD - 1

_sc_mesh = plsc.VectorSubcoreMesh(core_axis_name="c", subcore_axis_name="s")
_sc_params = pltpu.CompilerParams(use_tc_tiling_on_sc=False)


def _zero_acc_slice(buf_v, acc, sid):
    zero = jnp.zeros((LANES,), jnp.float32)

    @pl.loop(0, CHUNK)
    def _(i):
        buf_v[i, :] = zero

    @pl.loop(0, RPS // CHUNK)
    def _(j):
        pltpu.sync_copy(buf_v, acc.at[pl.ds(sid * RPS + j * CHUNK, CHUNK)])


def _count_body(dst_hbm, out_hbm, buf_v, didx_v, sem, acc):
    cid = lax.axis_index("c")
    sid = lax.axis_index("s")
    gw = cid * NS + sid

    _zero_acc_slice(buf_v, acc, sid)
    one = jnp.ones((LANES,), jnp.float32)

    @pl.loop(0, CHUNK)
    def _(i):
        buf_v[i, :] = one

    pltpu.sync_copy(dst_hbm.at[gw], didx_v)
    plsc.subcore_barrier()

    @pl.loop(0, K)
    def _(k):
        pltpu.sync_copy(buf_v, acc.at[didx_v.at[k]], add=True)

    plsc.subcore_barrier()
    pltpu.sync_copy(acc.at[pl.ds(sid * RPS, RPS)],
                    out_hbm.at[cid].at[pl.ds(sid * RPS, RPS)])


def _agg_body(h_hbm, src_hbm, dst_hbm, out_hbm,
              buf_v, sidx_v, didx_v, gbuf, sem, acc):
    cid = lax.axis_index("c")
    sid = lax.axis_index("s")
    gw = cid * NS + sid

    _zero_acc_slice(buf_v, acc, sid)
    pltpu.sync_copy(src_hbm.at[gw], sidx_v)
    pltpu.sync_copy(dst_hbm.at[gw], didx_v)
    plsc.subcore_barrier()

    @pl.loop(0, K)
    def _(k):
        pltpu.async_copy(h_hbm.at[sidx_v.at[k]], gbuf, sem).wait()
        pltpu.sync_copy(gbuf, acc.at[didx_v.at[k]], add=True)

    plsc.subcore_barrier()
    pltpu.sync_copy(acc.at[pl.ds(sid * RPS, RPS)],
                    out_hbm.at[cid].at[pl.ds(sid * RPS, RPS)])


@functools.partial(
    pl.kernel,
    out_type=jax.ShapeDtypeStruct((NC, N_PAD, LANES), jnp.float32),
    mesh=_sc_mesh,
    scratch_types=[
        pltpu.VMEM((CHUNK, LANES), jnp.float32),
        pltpu.VMEM((K, CHUNK), jnp.int32),
        pltpu.SemaphoreType.DMA,
        pltpu.VMEM_SHARED((N_PAD, LANES), jnp.float32),
    ],
    compiler_params=_sc_params,
)
def _sc_count(dst_hbm, out_hbm, buf_v, didx_v, sem, acc):
    _count_body(dst_hbm, out_hbm, buf_v, didx_v, sem, acc)


@functools.partial(
    pl.kernel,
    out_type=jax.ShapeDtypeStruct((NC, N_PAD, LANES), jnp.float32),
    mesh=_sc_mesh,
    scratch_types=[
        pltpu.VMEM((CHUNK, LANES), jnp.float32),
        pltpu.VMEM((K, CHUNK), jnp.int32),
        pltpu.VMEM((K, CHUNK), jnp.int32),
        pltpu.VMEM((CHUNK, LANES), jnp.float32),
        pltpu.SemaphoreType.DMA,
        pltpu.VMEM_SHARED((N_PAD, LANES), jnp.float32),
    ],
    compiler_params=_sc_params,
)
def _sc_agg(h_hbm, src_hbm, dst_hbm, out_hbm,
            buf_v, sidx_v, didx_v, gbuf, sem, acc):
    _agg_body(h_hbm, src_hbm, dst_hbm, out_hbm,
              buf_v, sidx_v, didx_v, gbuf, sem, acc)


MM_BLK = 2048
ROW_BLK = 2048


def _mm_body(x_ref, w_ref, o_ref):
    o_ref[...] = jnp.dot(x_ref[...], w_ref[...],
                         preferred_element_type=jnp.float32)


def _tc_matmul(x_pad, W1):
    return pl.pallas_call(
        _mm_body,
        grid=(N_PAD // MM_BLK,),
        in_specs=[pl.BlockSpec((MM_BLK, IN_DIM), lambda i: (i, 0)),
                  pl.BlockSpec((IN_DIM, HID), lambda i: (0, 0))],
        out_specs=pl.BlockSpec((MM_BLK, HID), lambda i: (i, 0)),
        out_shape=jax.ShapeDtypeStruct((N_PAD, HID), jnp.float32),
    )(x_pad, W1)


def _scale_body(cnt_ref, h_ref, dinv_ref, hp_ref):
    deg = cnt_ref[0] + cnt_ref[1] + 1.0
    dinv = lax.rsqrt(deg)
    dinv_ref[...] = dinv
    hp_ref[...] = h_ref[...] * dinv


def _tc_scale(cnt, h1):
    return pl.pallas_call(
        _scale_body,
        grid=(N_PAD // ROW_BLK,),
        in_specs=[pl.BlockSpec((NC, ROW_BLK, LANES), lambda i: (0, i, 0)),
                  pl.BlockSpec((ROW_BLK, LANES), lambda i: (i, 0))],
        out_specs=[pl.BlockSpec((ROW_BLK, LANES), lambda i: (i, 0)),
                   pl.BlockSpec((ROW_BLK, LANES), lambda i: (i, 0))],
        out_shape=[jax.ShapeDtypeStruct((N_PAD, LANES), jnp.float32),
                   jax.ShapeDtypeStruct((N_PAD, LANES), jnp.float32)],
    )(cnt, h1)


def _mid_body(p_ref, hp_ref, dinv_ref, w2_ref, b1_ref, h2p_ref):
    agg = p_ref[0] + p_ref[1] + hp_ref[...]
    z = jnp.maximum(dinv_ref[...] * agg + b1_ref[...], 0.0)
    h2 = jnp.dot(z, w2_ref[...], preferred_element_type=jnp.float32)
    h2p_ref[...] = h2 * dinv_ref[...]


def _tc_mid(p, h1p, dinv, W2p, b1):
    return pl.pallas_call(
        _mid_body,
        grid=(N_PAD // ROW_BLK,),
        in_specs=[pl.BlockSpec((NC, ROW_BLK, LANES), lambda i: (0, i, 0)),
                  pl.BlockSpec((ROW_BLK, LANES), lambda i: (i, 0)),
                  pl.BlockSpec((ROW_BLK, LANES), lambda i: (i, 0)),
                  pl.BlockSpec((LANES, LANES), lambda i: (0, 0)),
                  pl.BlockSpec((1, LANES), lambda i: (0, 0))],
        out_specs=pl.BlockSpec((ROW_BLK, LANES), lambda i: (i, 0)),
        out_shape=jax.ShapeDtypeStruct((N_PAD, LANES), jnp.float32),
    )(p, h1p, dinv, W2p, b1)


def _out_body(q_ref, h2p_ref, dinv_ref, b2_ref, o_ref):
    o = dinv_ref[...] * (q_ref[0] + q_ref[1] + h2p_ref[...]) + b2_ref[...]
    col = lax.broadcasted_iota(jnp.int32, o.shape, 1)
    valid = col < OUT_DIM
    masked = jnp.where(valid, o, -1e30)
    m = jnp.max(masked, axis=1, keepdims=True)
    e = jnp.where(valid, jnp.exp(o - m), 0.0)
    lse = jnp.log(jnp.sum(e, axis=1, keepdims=True)) + m
    o_ref[...] = o - lse


def _tc_out(q, h2p, dinv, b2p):
    return pl.pallas_call(
        _out_body,
        grid=(N_PAD // ROW_BLK,),
        in_specs=[pl.BlockSpec((NC, ROW_BLK, LANES), lambda i: (0, i, 0)),
                  pl.BlockSpec((ROW_BLK, LANES), lambda i: (i, 0)),
                  pl.BlockSpec((ROW_BLK, LANES), lambda i: (i, 0)),
                  pl.BlockSpec((1, LANES), lambda i: (0, 0))],
        out_specs=pl.BlockSpec((ROW_BLK, LANES), lambda i: (i, 0)),
        out_shape=jax.ShapeDtypeStruct((N_PAD, LANES), jnp.float32),
    )(q, h2p, dinv, b2p)


@jax.jit
def kernel(x, edge_index, W1, b1, W2, b2):
    ei = edge_index.astype(jnp.int32)
    src = jnp.concatenate(
        [ei[0], jnp.zeros((E_PAD - N_EDGES,), jnp.int32)]).reshape(NW, K, CHUNK)
    dst = jnp.concatenate(
        [ei[1], jnp.full((E_PAD - N_EDGES,), TRASH, jnp.int32)]).reshape(
            NW, K, CHUNK)

    x_pad = jnp.zeros((N_PAD, IN_DIM), jnp.float32).at[:N_NODES].set(x)
    W2p = jnp.zeros((LANES, LANES), jnp.float32).at[:HID, :OUT_DIM].set(W2)
    b1r = b1.reshape(1, LANES)
    b2p = jnp.zeros((1, LANES), jnp.float32).at[0, :OUT_DIM].set(b2)

    cnt = _sc_count(dst)
    h1 = _tc_matmul(x_pad, W1)
    dinv, h1p = _tc_scale(cnt, h1)

    p = _sc_agg(h1p, src, dst)
    h2p = _tc_mid(p, h1p, dinv, W2p, b1r)

    q = _sc_agg(h2p, src, dst)
    out = _tc_out(q, h2p, dinv, b2p)
    return out[:N_NODES, :OUT_DIM]


# speedup vs baseline: 39.3138x; 1.1741x over previous
import functools

import jax
import jax.numpy as jnp
from jax import lax
from jax.experimental import pallas as pl
from jax.experimental.pallas import tpu as pltpu
from jax.experimental.pallas import tpu_sc as plsc

N_NODES = 10000
N_EDGES = 320000
IN_DIM = 128
HID = 16
OUT_DIM = 7

NC = 2
NS = 16
NW = NC * NS
LANES = 16
CHUNK = 128
K = 80
E_PAD = NW * K * CHUNK
N_PAD = 10240
RPS = N_PAD // NS
TRASH = N_PAD - 1

_sc_mesh = plsc.VectorSubcoreMesh(core_axis_name="c", subcore_axis_name="s")
_sc_params = pltpu.CompilerParams(use_tc_tiling_on_sc=False)


def _zero_acc_slice(buf_v, acc, sid):
    zero = jnp.zeros((LANES,), jnp.float32)

    @pl.loop(0, CHUNK)
    def _(i):
        buf_v[i, :] = zero

    @pl.loop(0, RPS // CHUNK)
    def _(j):
        pltpu.sync_copy(buf_v, acc.at[pl.ds(sid * RPS + j * CHUNK, CHUNK)])


def _count_body(dst_hbm, out_hbm, buf_v, didx_v, sem, acc):
    cid = lax.axis_index("c")
    sid = lax.axis_index("s")
    gw = cid * NS + sid

    _zero_acc_slice(buf_v, acc, sid)
    one = jnp.ones((LANES,), jnp.float32)

    @pl.loop(0, CHUNK)
    def _(i):
        buf_v[i, :] = one

    pltpu.sync_copy(dst_hbm.at[gw], didx_v)
    plsc.subcore_barrier()

    @pl.loop(0, K)
    def _(k):
        pltpu.sync_copy(buf_v, acc.at[didx_v.at[k]], add=True)

    plsc.subcore_barrier()
    pltpu.sync_copy(acc.at[pl.ds(sid * RPS, RPS)],
                    out_hbm.at[cid].at[pl.ds(sid * RPS, RPS)])


def _agg_body(h_hbm, src_hbm, dst_hbm, out_hbm,
              buf_v, sidx_v, didx_v, gbuf0, gbuf1, sem0, sem1, acc):
    cid = lax.axis_index("c")
    sid = lax.axis_index("s")
    gw = cid * NS + sid

    _zero_acc_slice(buf_v, acc, sid)
    pltpu.sync_copy(src_hbm.at[gw], sidx_v)
    pltpu.sync_copy(dst_hbm.at[gw], didx_v)
    plsc.subcore_barrier()

    pltpu.async_copy(h_hbm.at[sidx_v.at[0]], gbuf0, sem0)
    pltpu.async_copy(h_hbm.at[sidx_v.at[1]], gbuf1, sem1)

    @pl.loop(0, K - 2, step=2)
    def _(k):
        pltpu.make_async_copy(h_hbm.at[sidx_v.at[k]], gbuf0, sem0).wait()
        pltpu.sync_copy(gbuf0, acc.at[didx_v.at[k]], add=True)
        pltpu.async_copy(h_hbm.at[sidx_v.at[k + 2]], gbuf0, sem0)
        pltpu.make_async_copy(h_hbm.at[sidx_v.at[k + 1]], gbuf1, sem1).wait()
        pltpu.sync_copy(gbuf1, acc.at[didx_v.at[k + 1]], add=True)
        pltpu.async_copy(h_hbm.at[sidx_v.at[k + 3]], gbuf1, sem1)

    pltpu.make_async_copy(h_hbm.at[sidx_v.at[K - 2]], gbuf0, sem0).wait()
    pltpu.sync_copy(gbuf0, acc.at[didx_v.at[K - 2]], add=True)
    pltpu.make_async_copy(h_hbm.at[sidx_v.at[K - 1]], gbuf1, sem1).wait()
    pltpu.sync_copy(gbuf1, acc.at[didx_v.at[K - 1]], add=True)

    plsc.subcore_barrier()
    pltpu.sync_copy(acc.at[pl.ds(sid * RPS, RPS)],
                    out_hbm.at[cid].at[pl.ds(sid * RPS, RPS)])


@functools.partial(
    pl.kernel,
    out_type=jax.ShapeDtypeStruct((NC, N_PAD, LANES), jnp.float32),
    mesh=_sc_mesh,
    scratch_types=[
        pltpu.VMEM((CHUNK, LANES), jnp.float32),
        pltpu.VMEM((K, CHUNK), jnp.int32),
        pltpu.SemaphoreType.DMA,
        pltpu.VMEM_SHARED((N_PAD, LANES), jnp.float32),
    ],
    compiler_params=_sc_params,
)
def _sc_count(dst_hbm, out_hbm, buf_v, didx_v, sem, acc):
    _count_body(dst_hbm, out_hbm, buf_v, didx_v, sem, acc)


@functools.partial(
    pl.kernel,
    out_type=jax.ShapeDtypeStruct((NC, N_PAD, LANES), jnp.float32),
    mesh=_sc_mesh,
    scratch_types=[
        pltpu.VMEM((CHUNK, LANES), jnp.float32),
        pltpu.VMEM((K, CHUNK), jnp.int32),
        pltpu.VMEM((K, CHUNK), jnp.int32),
        pltpu.VMEM((CHUNK, LANES), jnp.float32),
        pltpu.VMEM((CHUNK, LANES), jnp.float32),
        pltpu.SemaphoreType.DMA,
        pltpu.SemaphoreType.DMA,
        pltpu.VMEM_SHARED((N_PAD, LANES), jnp.float32),
    ],
    compiler_params=_sc_params,
)
def _sc_agg(h_hbm, src_hbm, dst_hbm, out_hbm,
            buf_v, sidx_v, didx_v, gbuf0, gbuf1, sem0, sem1, acc):
    _agg_body(h_hbm, src_hbm, dst_hbm, out_hbm,
              buf_v, sidx_v, didx_v, gbuf0, gbuf1, sem0, sem1, acc)


MM_BLK = 2048
ROW_BLK = 2048


def _mm_body(x_ref, w_ref, o_ref):
    o_ref[...] = jnp.dot(x_ref[...], w_ref[...],
                         preferred_element_type=jnp.float32)


def _tc_matmul(x_pad, W1):
    return pl.pallas_call(
        _mm_body,
        grid=(N_PAD // MM_BLK,),
        in_specs=[pl.BlockSpec((MM_BLK, IN_DIM), lambda i: (i, 0)),
                  pl.BlockSpec((IN_DIM, HID), lambda i: (0, 0))],
        out_specs=pl.BlockSpec((MM_BLK, HID), lambda i: (i, 0)),
        out_shape=jax.ShapeDtypeStruct((N_PAD, HID), jnp.float32),
    )(x_pad, W1)


def _scale_body(cnt_ref, h_ref, dinv_ref, hp_ref):
    deg = cnt_ref[0] + cnt_ref[1] + 1.0
    dinv = lax.rsqrt(deg)
    dinv_ref[...] = dinv
    hp_ref[...] = h_ref[...] * dinv


def _tc_scale(cnt, h1):
    return pl.pallas_call(
        _scale_body,
        grid=(N_PAD // ROW_BLK,),
        in_specs=[pl.BlockSpec((NC, ROW_BLK, LANES), lambda i: (0, i, 0)),
                  pl.BlockSpec((ROW_BLK, LANES), lambda i: (i, 0))],
        out_specs=[pl.BlockSpec((ROW_BLK, LANES), lambda i: (i, 0)),
                   pl.BlockSpec((ROW_BLK, LANES), lambda i: (i, 0))],
        out_shape=[jax.ShapeDtypeStruct((N_PAD, LANES), jnp.float32),
                   jax.ShapeDtypeStruct((N_PAD, LANES), jnp.float32)],
    )(cnt, h1)


def _mid_body(p_ref, hp_ref, dinv_ref, w2_ref, b1_ref, h2p_ref):
    agg = p_ref[0] + p_ref[1] + hp_ref[...]
    z = jnp.maximum(dinv_ref[...] * agg + b1_ref[...], 0.0)
    h2 = jnp.dot(z, w2_ref[...], preferred_element_type=jnp.float32)
    h2p_ref[...] = h2 * dinv_ref[...]


def _tc_mid(p, h1p, dinv, W2p, b1):
    return pl.pallas_call(
        _mid_body,
        grid=(N_PAD // ROW_BLK,),
        in_specs=[pl.BlockSpec((NC, ROW_BLK, LANES), lambda i: (0, i, 0)),
                  pl.BlockSpec((ROW_BLK, LANES), lambda i: (i, 0)),
                  pl.BlockSpec((ROW_BLK, LANES), lambda i: (i, 0)),
                  pl.BlockSpec((LANES, LANES), lambda i: (0, 0)),
                  pl.BlockSpec((1, LANES), lambda i: (0, 0))],
        out_specs=pl.BlockSpec((ROW_BLK, LANES), lambda i: (i, 0)),
        out_shape=jax.ShapeDtypeStruct((N_PAD, LANES), jnp.float32),
    )(p, h1p, dinv, W2p, b1)


def _out_body(q_ref, h2p_ref, dinv_ref, b2_ref, o_ref):
    o = dinv_ref[...] * (q_ref[0] + q_ref[1] + h2p_ref[...]) + b2_ref[...]
    col = lax.broadcasted_iota(jnp.int32, o.shape, 1)
    valid = col < OUT_DIM
    masked = jnp.where(valid, o, -1e30)
    m = jnp.max(masked, axis=1, keepdims=True)
    e = jnp.where(valid, jnp.exp(o - m), 0.0)
    lse = jnp.log(jnp.sum(e, axis=1, keepdims=True)) + m
    o_ref[...] = o - lse


def _tc_out(q, h2p, dinv, b2p):
    return pl.pallas_call(
        _out_body,
        grid=(N_PAD // ROW_BLK,),
        in_specs=[pl.BlockSpec((NC, ROW_BLK, LANES), lambda i: (0, i, 0)),
                  pl.BlockSpec((ROW_BLK, LANES), lambda i: (i, 0)),
                  pl.BlockSpec((ROW_BLK, LANES), lambda i: (i, 0)),
                  pl.BlockSpec((1, LANES), lambda i: (0, 0))],
        out_specs=pl.BlockSpec((ROW_BLK, LANES), lambda i: (i, 0)),
        out_shape=jax.ShapeDtypeStruct((N_PAD, LANES), jnp.float32),
    )(q, h2p, dinv, b2p)


@jax.jit
def kernel(x, edge_index, W1, b1, W2, b2):
    ei = edge_index.astype(jnp.int32)
    src = jnp.concatenate(
        [ei[0], jnp.zeros((E_PAD - N_EDGES,), jnp.int32)]).reshape(NW, K, CHUNK)
    dst = jnp.concatenate(
        [ei[1], jnp.full((E_PAD - N_EDGES,), TRASH, jnp.int32)]).reshape(
            NW, K, CHUNK)

    x_pad = jnp.zeros((N_PAD, IN_DIM), jnp.float32).at[:N_NODES].set(x)
    W2p = jnp.zeros((LANES, LANES), jnp.float32).at[:HID, :OUT_DIM].set(W2)
    b1r = b1.reshape(1, LANES)
    b2p = jnp.zeros((1, LANES), jnp.float32).at[0, :OUT_DIM].set(b2)

    cnt = _sc_count(dst)
    h1 = _tc_matmul(x_pad, W1)
    dinv, h1p = _tc_scale(cnt, h1)

    p = _sc_agg(h1p, src, dst)
    h2p = _tc_mid(p, h1p, dinv, W2p, b1r)

    q = _sc_agg(h2p, src, dst)
    out = _tc_out(q, h2p, dinv, b2p)
    return out[:N_NODES, :OUT_DIM]


# speedup vs baseline: 41.1318x; 1.0462x over previous
import functools

import jax
import jax.numpy as jnp
from jax import lax
from jax.experimental import pallas as pl
from jax.experimental.pallas import tpu as pltpu
from jax.experimental.pallas import tpu_sc as plsc

N_NODES = 10000
N_EDGES = 320000
IN_DIM = 128
HID = 16
OUT_DIM = 7

NC = 2
NS = 16
NW = NC * NS
LANES = 16
CHUNK = 128
K = 80
E_PAD = NW * K * CHUNK
N_PAD = 10240
RPS = N_PAD // NS
TRASH = N_PAD - 1

_sc_mesh = plsc.VectorSubcoreMesh(core_axis_name="c", subcore_axis_name="s")
_sc_params = pltpu.CompilerParams(use_tc_tiling_on_sc=False)


def _zero_acc_slice(buf_v, acc, sid):
    zero = jnp.zeros((LANES,), jnp.float32)

    @pl.loop(0, CHUNK)
    def _(i):
        buf_v[i, :] = zero

    @pl.loop(0, RPS // CHUNK)
    def _(j):
        pltpu.sync_copy(buf_v, acc.at[pl.ds(sid * RPS + j * CHUNK, CHUNK)])


def _count_body(dst_hbm, out_hbm, buf_v, didx_v, sem, acc):
    cid = lax.axis_index("c")
    sid = lax.axis_index("s")
    gw = cid * NS + sid

    _zero_acc_slice(buf_v, acc, sid)
    one = jnp.ones((LANES,), jnp.float32)

    @pl.loop(0, CHUNK)
    def _(i):
        buf_v[i, :] = one

    pltpu.sync_copy(dst_hbm.at[gw], didx_v)
    plsc.subcore_barrier()

    @pl.loop(0, K)
    def _(k):
        pltpu.sync_copy(buf_v, acc.at[didx_v.at[k]], add=True)

    plsc.subcore_barrier()
    pltpu.sync_copy(acc.at[pl.ds(sid * RPS, RPS)],
                    out_hbm.at[cid].at[pl.ds(sid * RPS, RPS)])


NBUF = 4


def _agg_body(h_hbm, src_hbm, dst_hbm, out_hbm,
              buf_v, sidx_v, didx_v, gbufs, sems, acc):
    cid = lax.axis_index("c")
    sid = lax.axis_index("s")
    gw = cid * NS + sid

    _zero_acc_slice(buf_v, acc, sid)
    pltpu.sync_copy(src_hbm.at[gw], sidx_v)
    pltpu.sync_copy(dst_hbm.at[gw], didx_v)
    plsc.subcore_barrier()

    for b in range(NBUF):
        pltpu.async_copy(h_hbm.at[sidx_v.at[b]], gbufs[b], sems[b])

    @pl.loop(0, K - NBUF, step=NBUF)
    def _(k):
        for b in range(NBUF):
            pltpu.make_async_copy(
                h_hbm.at[sidx_v.at[k + b]], gbufs[b], sems[b]).wait()
            pltpu.sync_copy(gbufs[b], acc.at[didx_v.at[k + b]], add=True)
            pltpu.async_copy(
                h_hbm.at[sidx_v.at[k + NBUF + b]], gbufs[b], sems[b])

    for b in range(NBUF):
        pltpu.make_async_copy(
            h_hbm.at[sidx_v.at[K - NBUF + b]], gbufs[b], sems[b]).wait()
        pltpu.sync_copy(gbufs[b], acc.at[didx_v.at[K - NBUF + b]], add=True)

    plsc.subcore_barrier()
    pltpu.sync_copy(acc.at[pl.ds(sid * RPS, RPS)],
                    out_hbm.at[cid].at[pl.ds(sid * RPS, RPS)])


@functools.partial(
    pl.kernel,
    out_type=jax.ShapeDtypeStruct((NC, N_PAD, LANES), jnp.float32),
    mesh=_sc_mesh,
    scratch_types=[
        pltpu.VMEM((CHUNK, LANES), jnp.float32),
        pltpu.VMEM((K, CHUNK), jnp.int32),
        pltpu.SemaphoreType.DMA,
        pltpu.VMEM_SHARED((N_PAD, LANES), jnp.float32),
    ],
    compiler_params=_sc_params,
)
def _sc_count(dst_hbm, out_hbm, buf_v, didx_v, sem, acc):
    _count_body(dst_hbm, out_hbm, buf_v, didx_v, sem, acc)


@functools.partial(
    pl.kernel,
    out_type=jax.ShapeDtypeStruct((NC, N_PAD, LANES), jnp.float32),
    mesh=_sc_mesh,
    scratch_types=(
        [pltpu.VMEM((CHUNK, LANES), jnp.float32),
         pltpu.VMEM((K, CHUNK), jnp.int32),
         pltpu.VMEM((K, CHUNK), jnp.int32)]
        + [pltpu.VMEM((CHUNK, LANES), jnp.float32)] * NBUF
        + [pltpu.SemaphoreType.DMA] * NBUF
        + [pltpu.VMEM_SHARED((N_PAD, LANES), jnp.float32)]
    ),
    compiler_params=_sc_params,
)
def _sc_agg(h_hbm, src_hbm, dst_hbm, out_hbm,
            buf_v, sidx_v, didx_v, *rest):
    gbufs = rest[:NBUF]
    sems = rest[NBUF:2 * NBUF]
    acc = rest[2 * NBUF]
    _agg_body(h_hbm, src_hbm, dst_hbm, out_hbm,
              buf_v, sidx_v, didx_v, gbufs, sems, acc)


MM_BLK = 2048
ROW_BLK = 2048


def _mm_body(x_ref, w_ref, o_ref):
    o_ref[...] = jnp.dot(x_ref[...], w_ref[...],
                         preferred_element_type=jnp.float32)


def _tc_matmul(x_pad, W1):
    return pl.pallas_call(
        _mm_body,
        grid=(N_PAD // MM_BLK,),
        in_specs=[pl.BlockSpec((MM_BLK, IN_DIM), lambda i: (i, 0)),
                  pl.BlockSpec((IN_DIM, HID), lambda i: (0, 0))],
        out_specs=pl.BlockSpec((MM_BLK, HID), lambda i: (i, 0)),
        out_shape=jax.ShapeDtypeStruct((N_PAD, HID), jnp.float32),
    )(x_pad, W1)


def _scale_body(cnt_ref, h_ref, dinv_ref, hp_ref):
    deg = cnt_ref[0] + cnt_ref[1] + 1.0
    dinv = lax.rsqrt(deg)
    dinv_ref[...] = dinv
    hp_ref[...] = h_ref[...] * dinv


def _tc_scale(cnt, h1):
    return pl.pallas_call(
        _scale_body,
        grid=(N_PAD // ROW_BLK,),
        in_specs=[pl.BlockSpec((NC, ROW_BLK, LANES), lambda i: (0, i, 0)),
                  pl.BlockSpec((ROW_BLK, LANES), lambda i: (i, 0))],
        out_specs=[pl.BlockSpec((ROW_BLK, LANES), lambda i: (i, 0)),
                   pl.BlockSpec((ROW_BLK, LANES), lambda i: (i, 0))],
        out_shape=[jax.ShapeDtypeStruct((N_PAD, LANES), jnp.float32),
                   jax.ShapeDtypeStruct((N_PAD, LANES), jnp.float32)],
    )(cnt, h1)


def _mid_body(p_ref, hp_ref, dinv_ref, w2_ref, b1_ref, h2p_ref):
    agg = p_ref[0] + p_ref[1] + hp_ref[...]
    z = jnp.maximum(dinv_ref[...] * agg + b1_ref[...], 0.0)
    h2 = jnp.dot(z, w2_ref[...], preferred_element_type=jnp.float32)
    h2p_ref[...] = h2 * dinv_ref[...]


def _tc_mid(p, h1p, dinv, W2p, b1):
    return pl.pallas_call(
        _mid_body,
        grid=(N_PAD // ROW_BLK,),
        in_specs=[pl.BlockSpec((NC, ROW_BLK, LANES), lambda i: (0, i, 0)),
                  pl.BlockSpec((ROW_BLK, LANES), lambda i: (i, 0)),
                  pl.BlockSpec((ROW_BLK, LANES), lambda i: (i, 0)),
                  pl.BlockSpec((LANES, LANES), lambda i: (0, 0)),
                  pl.BlockSpec((1, LANES), lambda i: (0, 0))],
        out_specs=pl.BlockSpec((ROW_BLK, LANES), lambda i: (i, 0)),
        out_shape=jax.ShapeDtypeStruct((N_PAD, LANES), jnp.float32),
    )(p, h1p, dinv, W2p, b1)


def _out_body(q_ref, h2p_ref, dinv_ref, b2_ref, o_ref):
    o = dinv_ref[...] * (q_ref[0] + q_ref[1] + h2p_ref[...]) + b2_ref[...]
    col = lax.broadcasted_iota(jnp.int32, o.shape, 1)
    valid = col < OUT_DIM
    masked = jnp.where(valid, o, -1e30)
    m = jnp.max(masked, axis=1, keepdims=True)
    e = jnp.where(valid, jnp.exp(o - m), 0.0)
    lse = jnp.log(jnp.sum(e, axis=1, keepdims=True)) + m
    o_ref[...] = o - lse


def _tc_out(q, h2p, dinv, b2p):
    return pl.pallas_call(
        _out_body,
        grid=(N_PAD // ROW_BLK,),
        in_specs=[pl.BlockSpec((NC, ROW_BLK, LANES), lambda i: (0, i, 0)),
                  pl.BlockSpec((ROW_BLK, LANES), lambda i: (i, 0)),
                  pl.BlockSpec((ROW_BLK, LANES), lambda i: (i, 0)),
                  pl.BlockSpec((1, LANES), lambda i: (0, 0))],
        out_specs=pl.BlockSpec((ROW_BLK, LANES), lambda i: (i, 0)),
        out_shape=jax.ShapeDtypeStruct((N_PAD, LANES), jnp.float32),
    )(q, h2p, dinv, b2p)


@jax.jit
def kernel(x, edge_index, W1, b1, W2, b2):
    ei = edge_index.astype(jnp.int32)
    src = jnp.concatenate(
        [ei[0], jnp.zeros((E_PAD - N_EDGES,), jnp.int32)]).reshape(NW, K, CHUNK)
    dst = jnp.concatenate(
        [ei[1], jnp.full((E_PAD - N_EDGES,), TRASH, jnp.int32)]).reshape(
            NW, K, CHUNK)

    x_pad = jnp.zeros((N_PAD, IN_DIM), jnp.float32).at[:N_NODES].set(x)
    W2p = jnp.zeros((LANES, LANES), jnp.float32).at[:HID, :OUT_DIM].set(W2)
    b1r = b1.reshape(1, LANES)
    b2p = jnp.zeros((1, LANES), jnp.float32).at[0, :OUT_DIM].set(b2)

    cnt = _sc_count(dst)
    h1 = _tc_matmul(x_pad, W1)
    dinv, h1p = _tc_scale(cnt, h1)

    p = _sc_agg(h1p, src, dst)
    h2p = _tc_mid(p, h1p, dinv, W2p, b1r)

    q = _sc_agg(h2p, src, dst)
    out = _tc_out(q, h2p, dinv, b2p)
    return out[:N_NODES, :OUT_DIM]
